# baseline (speedup 1.0000x reference)
import jax
import jax.numpy as jnp
from jax.experimental import pallas as pl
from jax.experimental.pallas import tpu as pltpu
from jax.experimental.pallas import tpu_sc as plsc

_T = 8192
_D = 1024
_O = 1024
_E = 8

_RB = 1024
_B = 256
_NBLK = _T // _B + _E
_CAP = _NBLK * _B

_C = 64
_W = 16


def _router_kernel(x_ref, wr1_ref, br1_ref, wr2_ref, br2_ref, mask_ref):
    h = jnp.dot(x_ref[...], wr1_ref[...], preferred_element_type=jnp.float32)
    h = jnp.maximum(h + br1_ref[...], 0.0)
    s = jnp.dot(h, wr2_ref[...], preferred_element_type=jnp.float32)
    s = s + br2_ref[...]
    m = jnp.max(s, axis=-1, keepdims=True)
    iota = jax.lax.broadcasted_iota(jnp.int32, s.shape, 1)
    am = jnp.min(jnp.where(s == m, iota, _E), axis=-1, keepdims=True)
    mask_ref[...] = (iota == am).astype(jnp.float32)


def _dispatch_kernel(oh_ref, pos_ref, blk_ref):
    oh = oh_ref[...]

    R = jax.lax.broadcasted_iota(jnp.int32, (8 * 128, 8 * 128), 0)
    Cl = jax.lax.broadcasted_iota(jnp.int32, (8 * 128, 8 * 128), 1)
    W1 = ((R % _E == Cl % _E) & (R // _E < Cl // _E)).astype(jnp.float32)
    Tr = jax.lax.broadcasted_iota(jnp.int32, (8 * 128, _E), 0)
    Te = jax.lax.broadcasted_iota(jnp.int32, (8 * 128, _E), 1)
    T8 = (Tr % _E == Te).astype(jnp.float32)
    Ecols = jax.lax.broadcasted_iota(jnp.int32, (_E, 8 * 128), 1)
    Erows = jax.lax.broadcasted_iota(jnp.int32, (_E, 8 * 128), 0)
    E8 = (Ecols % _E == Erows).astype(jnp.float32)
    cc = jax.lax.broadcasted_iota(jnp.int32, (_C, _C), 0)
    cp = jax.lax.broadcasted_iota(jnp.int32, (_C, _C), 1)
    A = (cp < cc).astype(jnp.float32)
    Srow = jax.lax.broadcasted_iota(jnp.int32, (8 * 128, 128), 0)
    Scol = jax.lax.broadcasted_iota(jnp.int32, (8 * 128, 128), 1)
    S = (Srow // _E == Scol).astype(jnp.float32)

    within = jnp.dot(oh, W1, preferred_element_type=jnp.float32)
    tot = jnp.dot(oh, T8, preferred_element_type=jnp.float32)
    prefix = jnp.dot(A, tot, preferred_element_type=jnp.float32)
    prefixI = jnp.dot(prefix, E8, preferred_element_type=jnp.float32)
    rank = within + prefixI

    base_row = jnp.zeros((1, 8 * 128), jnp.float32)
    cum = jnp.zeros((1, 1), jnp.float32)
    cum_list = []
    for e in range(_E):
        cnt = jnp.sum(tot[:, e:e + 1]).reshape(1, 1)
        nblk = jnp.floor((cnt + (_B - 1)) * (1.0 / _B))
        base_row = base_row + (cum * _B) * E8[e:e + 1, :]
        cum = cum + nblk
        cum_list.append(cum)

    posI = oh * (rank + base_row)
    pos2d = jnp.dot(posI, S, preferred_element_type=jnp.float32)
    pos_ref[...] = pos2d.astype(jnp.int32)

    gi = jax.lax.broadcasted_iota(jnp.int32, (1, _C), 1).astype(jnp.float32)
    be = jnp.zeros((1, _C), jnp.float32)
    for e in range(_E - 1):
        be = be + (gi >= cum_list[e]).astype(jnp.float32)
    blk_ref[...] = be.astype(jnp.int32)


def _expert_kernel(be_ref, xs_ref, w_ref, b_ref, out_ref):
    del be_ref
    acc = jnp.dot(xs_ref[...], w_ref[...], preferred_element_type=jnp.float32)
    out_ref[...] = acc + b_ref[0]


_NC = 2
_NS = 16
_NW = _NC * _NS
_PER = _T // _NW
_CH = 64


def _vector_mesh():
    return plsc.VectorSubcoreMesh(core_axis_name="c", subcore_axis_name="s")


def _sc_scatter(x, pos):
    @pl.kernel(out_type=jax.ShapeDtypeStruct((_CAP, _D), x.dtype),
               mesh=_vector_mesh(),
               scratch_types=[pltpu.VMEM((_CH,), jnp.int32),
                              pltpu.VMEM((_CH, _D), jnp.float32),
                              pltpu.SemaphoreType.DMA])
    def run(x_hbm, i_hbm, o_hbm, idx_v, rows_v, sem):
        wid = jax.lax.axis_index("s") * _NC + jax.lax.axis_index("c")
        base = wid * _PER
        for t in range(_PER // _CH):
            off = base + t * _CH
            pltpu.sync_copy(i_hbm.at[pl.ds(off, _CH)], idx_v)
            pltpu.sync_copy(x_hbm.at[pl.ds(off, _CH)], rows_v)
            pltpu.async_copy(rows_v, o_hbm.at[idx_v], sem).wait()

    return run(x, pos)


def _sc_gather(src, pos):
    @pl.kernel(out_type=jax.ShapeDtypeStruct((_T, _D), src.dtype),
               mesh=_vector_mesh(),
               scratch_types=[pltpu.VMEM((_CH,), jnp.int32),
                              pltpu.VMEM((_CH, _D), jnp.float32),
                              pltpu.SemaphoreType.DMA])
    def run(src_hbm, i_hbm, o_hbm, idx_v, rows_v, sem):
        wid = jax.lax.axis_index("s") * _NC + jax.lax.axis_index("c")
        base = wid * _PER
        for t in range(_PER // _CH):
            off = base + t * _CH
            pltpu.sync_copy(i_hbm.at[pl.ds(off, _CH)], idx_v)
            pltpu.async_copy(src_hbm.at[idx_v], rows_v, sem).wait()
            pltpu.sync_copy(rows_v, o_hbm.at[pl.ds(off, _CH)])

    return run(src, pos)


@jax.jit
def kernel(x, W_exp, b_exp, W_r1, b_r1, W_r2, b_r2):
    mask = pl.pallas_call(
        _router_kernel,
        grid=(_T // _RB,),
        in_specs=[
            pl.BlockSpec((_RB, _D), lambda i: (i, 0)),
            pl.BlockSpec((_D, 2 * _D), lambda i: (0, 0)),
            pl.BlockSpec((1, 2 * _D), lambda i: (0, 0)),
            pl.BlockSpec((2 * _D, _E), lambda i: (0, 0)),
            pl.BlockSpec((1, _E), lambda i: (0, 0)),
        ],
        out_specs=pl.BlockSpec((_RB, _E), lambda i: (i, 0)),
        out_shape=jax.ShapeDtypeStruct((_T, _E), jnp.float32),
    )(x, W_r1, b_r1.reshape(1, -1), W_r2, b_r2.reshape(1, -1))

    oh2 = mask.reshape(_C, 8 * 128)
    pos2d, blk64 = pl.pallas_call(
        _dispatch_kernel,
        grid=(1,),
        in_specs=[pl.BlockSpec((_C, 8 * 128), lambda i: (0, 0))],
        out_specs=[pl.BlockSpec((_C, 128), lambda i: (0, 0)),
                   pl.BlockSpec((1, _C), lambda i: (0, 0))],
        out_shape=[jax.ShapeDtypeStruct((_C, 128), jnp.int32),
                   jax.ShapeDtypeStruct((1, _C), jnp.int32)],
    )(oh2)
    pos = pos2d.reshape(_T)
    blk_exp = blk64.reshape(-1)[:_NBLK]

    x_sorted = _sc_scatter(x, pos)

    out_sorted = pl.pallas_call(
        _expert_kernel,
        grid_spec=pltpu.PrefetchScalarGridSpec(
            num_scalar_prefetch=1,
            grid=(_NBLK,),
            in_specs=[
                pl.BlockSpec((_B, _D), lambda g, be: (g, 0)),
                pl.BlockSpec((_D, _O), lambda g, be: (0, be[g])),
                pl.BlockSpec((1, 1, _O), lambda g, be: (be[g], 0, 0)),
            ],
            out_specs=pl.BlockSpec((_B, _O), lambda g, be: (g, 0)),
        ),
        out_shape=jax.ShapeDtypeStruct((_CAP, _O), jnp.float32),
    )(blk_exp, x_sorted, W_exp, b_exp.reshape(_E, 1, _O))

    return _sc_gather(out_sorted, pos)


# speedup vs baseline: 1.1008x; 1.1008x over previous
import jax
import jax.numpy as jnp
from jax.experimental import pallas as pl
from jax.experimental.pallas import tpu as pltpu
from jax.experimental.pallas import tpu_sc as plsc

_T = 8192
_D = 1024
_O = 1024
_E = 8

_RB = 1024
_B = 256
_NBLK = _T // _B + _E
_CAP = _NBLK * _B

_C = 64
_W = 16


def _router_kernel(x_ref, wr1_ref, br1_ref, wr2_ref, br2_ref, s_ref):
    h = jnp.dot(x_ref[...], wr1_ref[...], preferred_element_type=jnp.float32)
    h = jnp.maximum(h + br1_ref[...], 0.0)
    s = jnp.dot(h, wr2_ref[...], preferred_element_type=jnp.float32)
    s_ref[...] = s + br2_ref[...]


def _dispatch_kernel(s3_ref, pos_ref, blk_ref):
    s = [s3_ref[e] for e in range(_E)]
    m = s[0]
    for e in range(1, _E):
        m = jnp.maximum(m, s[e])
    eidx = jnp.full((_C, 128), _E - 1, jnp.int32)
    for e in range(_E - 2, -1, -1):
        eidx = jnp.where(s[e] == m, e, eidx)

    jj = jax.lax.broadcasted_iota(jnp.int32, (128, 128), 0)
    rr = jax.lax.broadcasted_iota(jnp.int32, (128, 128), 1)
    U = (jj < rr).astype(jnp.float32)
    cc = jax.lax.broadcasted_iota(jnp.int32, (_C, _C), 0)
    cp = jax.lax.broadcasted_iota(jnp.int32, (_C, _C), 1)
    A = (cp < cc).astype(jnp.float32)

    pos = jnp.zeros((_C, 128), jnp.float32)
    cum = jnp.zeros((1, 1), jnp.float32)
    cum_list = []
    for e in range(_E):
        oh = (eidx == e).astype(jnp.float32)
        within = jnp.dot(oh, U, preferred_element_type=jnp.float32)
        tot = jnp.sum(oh, axis=1, keepdims=True)
        prefix = jnp.dot(A, tot, preferred_element_type=jnp.float32)
        rank = within + prefix
        cnt = jnp.sum(tot).reshape(1, 1)
        nblk = jnp.floor((cnt + (_B - 1)) * (1.0 / _B))
        pos = pos + oh * (cum * _B + rank)
        cum = cum + nblk
        cum_list.append(cum)
    pos_ref[...] = pos.astype(jnp.int32)

    gi = jax.lax.broadcasted_iota(jnp.int32, (1, _C), 1).astype(jnp.float32)
    be = jnp.zeros((1, _C), jnp.float32)
    for e in range(_E - 1):
        be = be + (gi >= cum_list[e]).astype(jnp.float32)
    blk_ref[...] = be.astype(jnp.int32)


def _expert_kernel(be_ref, xs_ref, w_ref, b_ref, out_ref):
    del be_ref
    acc = jnp.dot(xs_ref[...], w_ref[...], preferred_element_type=jnp.float32)
    out_ref[...] = acc + b_ref[0]


_NC = 2
_NS = 16
_NW = _NC * _NS
_PER = _T // _NW
_CH = 32
_NCH = _PER // _CH


def _vector_mesh():
    return plsc.VectorSubcoreMesh(core_axis_name="c", subcore_axis_name="s")


_SC_SCRATCH = [pltpu.VMEM((_NCH, _CH), jnp.int32),
               pltpu.VMEM((2, _CH, _D), jnp.float32),
               pltpu.SemaphoreType.DMA,
               pltpu.SemaphoreType.DMA,
               pltpu.SemaphoreType.DMA,
               pltpu.SemaphoreType.DMA]


def _sc_scatter(x, pos):
    @pl.kernel(out_type=jax.ShapeDtypeStruct((_CAP, _D), x.dtype),
               mesh=_vector_mesh(), scratch_types=_SC_SCRATCH)
    def run(x_hbm, i_hbm, o_hbm, idx_v, rows_v, l0, l1, s0, s1):
        wid = jax.lax.axis_index("s") * _NC + jax.lax.axis_index("c")
        base = wid * _PER
        lsem = [l0, l1]
        ssem = [s0, s1]
        ld = [pltpu.async_copy(x_hbm.at[pl.ds(base + t * _CH, _CH)],
                               rows_v.at[t], lsem[t]) for t in range(2)]
        pltpu.sync_copy(i_hbm.at[pl.ds(wid * _NCH, _NCH)], idx_v)
        sc = [None, None]
        for t in range(_NCH):
            b = t % 2
            ld[b].wait()
            sc[b] = pltpu.async_copy(rows_v.at[b], o_hbm.at[idx_v.at[t]],
                                     ssem[b])
            if 1 <= t < _NCH - 1:
                sc[1 - b].wait()
                ld[1 - b] = pltpu.async_copy(
                    x_hbm.at[pl.ds(base + (t + 1) * _CH, _CH)],
                    rows_v.at[1 - b], lsem[1 - b])
        sc[_NCH % 2].wait()
        sc[1 - _NCH % 2].wait()

    return run(x, pos)


def _sc_gather(src, pos):
    @pl.kernel(out_type=jax.ShapeDtypeStruct((_T, _D), src.dtype),
               mesh=_vector_mesh(), scratch_types=_SC_SCRATCH)
    def run(src_hbm, i_hbm, o_hbm, idx_v, rows_v, g0, g1, t0, t1):
        wid = jax.lax.axis_index("s") * _NC + jax.lax.axis_index("c")
        base = wid * _PER
        gsem = [g0, g1]
        tsem = [t0, t1]
        pltpu.sync_copy(i_hbm.at[pl.ds(wid * _NCH, _NCH)], idx_v)
        ga = [pltpu.async_copy(src_hbm.at[idx_v.at[t]], rows_v.at[t],
                               gsem[t]) for t in range(2)]
        st = [None, None]
        for t in range(_NCH):
            b = t % 2
            ga[b].wait()
            st[b] = pltpu.async_copy(rows_v.at[b],
                                     o_hbm.at[pl.ds(base + t * _CH, _CH)],
                                     tsem[b])
            if 1 <= t < _NCH - 1:
                st[1 - b].wait()
                ga[1 - b] = pltpu.async_copy(src_hbm.at[idx_v.at[t + 1]],
                                             rows_v.at[1 - b], gsem[1 - b])
        st[_NCH % 2].wait()
        st[1 - _NCH % 2].wait()

    return run(src, pos)


@jax.jit
def kernel(x, W_exp, b_exp, W_r1, b_r1, W_r2, b_r2):
    scores = pl.pallas_call(
        _router_kernel,
        grid=(_T // _RB,),
        in_specs=[
            pl.BlockSpec((_RB, _D), lambda i: (i, 0)),
            pl.BlockSpec((_D, 2 * _D), lambda i: (0, 0)),
            pl.BlockSpec((1, 2 * _D), lambda i: (0, 0)),
            pl.BlockSpec((2 * _D, _E), lambda i: (0, 0)),
            pl.BlockSpec((1, _E), lambda i: (0, 0)),
        ],
        out_specs=pl.BlockSpec((_RB, _E), lambda i: (i, 0)),
        out_shape=jax.ShapeDtypeStruct((_T, _E), jnp.float32),
    )(x, W_r1, b_r1.reshape(1, -1), W_r2, b_r2.reshape(1, -1))

    s3 = scores.T.reshape(_E, _C, 128)
    pos2d, blk64 = pl.pallas_call(
        _dispatch_kernel,
        grid=(1,),
        in_specs=[pl.BlockSpec((_E, _C, 128), lambda i: (0, 0, 0))],
        out_specs=[pl.BlockSpec((_C, 128), lambda i: (0, 0)),
                   pl.BlockSpec((1, _C), lambda i: (0, 0))],
        out_shape=[jax.ShapeDtypeStruct((_C, 128), jnp.int32),
                   jax.ShapeDtypeStruct((1, _C), jnp.int32)],
    )(s3)
    pos = pos2d.reshape(_T // _CH, _CH)
    blk_exp = blk64.reshape(-1)[:_NBLK]

    x_sorted = _sc_scatter(x, pos)

    out_sorted = pl.pallas_call(
        _expert_kernel,
        grid_spec=pltpu.PrefetchScalarGridSpec(
            num_scalar_prefetch=1,
            grid=(_NBLK,),
            in_specs=[
                pl.BlockSpec((_B, _D), lambda g, be: (g, 0)),
                pl.BlockSpec((_D, _O), lambda g, be: (0, be[g])),
                pl.BlockSpec((1, 1, _O), lambda g, be: (be[g], 0, 0)),
            ],
            out_specs=pl.BlockSpec((_B, _O), lambda g, be: (g, 0)),
        ),
        out_shape=jax.ShapeDtypeStruct((_CAP, _O), jnp.float32),
    )(blk_exp, x_sorted, W_exp, b_exp.reshape(_E, 1, _O))

    return _sc_gather(out_sorted, pos)


# speedup vs baseline: 1.1230x; 1.0202x over previous
import jax
import jax.numpy as jnp
from jax.experimental import pallas as pl
from jax.experimental.pallas import tpu as pltpu
from jax.experimental.pallas import tpu_sc as plsc

_T = 8192
_D = 1024
_O = 1024
_E = 8

_RB = 1024
_B = 256
_NBLK = _T // _B + _E
_CAP = _NBLK * _B

_C = 64
_W = 16


def _router_kernel(x_ref, wr1_ref, br1_ref, wr2_ref, br2_ref, s_ref):
    h = jnp.dot(x_ref[...], wr1_ref[...], preferred_element_type=jnp.float32)
    h = jnp.maximum(h + br1_ref[...], 0.0)
    s = jnp.dot(h, wr2_ref[...], preferred_element_type=jnp.float32)
    s_ref[...] = jnp.transpose(s + br2_ref[...])


def _dispatch_kernel(s3_ref, pos_ref, blk_ref):
    s = [s3_ref[e] for e in range(_E)]
    m = s[0]
    for e in range(1, _E):
        m = jnp.maximum(m, s[e])
    eidx = jnp.full((_C, 128), _E - 1, jnp.int32)
    for e in range(_E - 2, -1, -1):
        eidx = jnp.where(s[e] == m, e, eidx)

    jj = jax.lax.broadcasted_iota(jnp.int32, (128, 128), 0)
    rr = jax.lax.broadcasted_iota(jnp.int32, (128, 128), 1)
    U = (jj < rr).astype(jnp.float32)
    cc = jax.lax.broadcasted_iota(jnp.int32, (_C, _C), 0)
    cp = jax.lax.broadcasted_iota(jnp.int32, (_C, _C), 1)
    A = (cp < cc).astype(jnp.float32)

    pos = jnp.zeros((_C, 128), jnp.float32)
    cum = jnp.zeros((1, 1), jnp.float32)
    cum_list = []
    for e in range(_E):
        oh = (eidx == e).astype(jnp.float32)
        within = jnp.dot(oh, U, preferred_element_type=jnp.float32)
        tot = jnp.sum(oh, axis=1, keepdims=True)
        prefix = jnp.dot(A, tot, preferred_element_type=jnp.float32)
        rank = within + prefix
        cnt = jnp.sum(tot).reshape(1, 1)
        nblk = jnp.floor((cnt + (_B - 1)) * (1.0 / _B))
        pos = pos + oh * (cum * _B + rank)
        cum = cum + nblk
        cum_list.append(cum)
    pos_ref[...] = pos.astype(jnp.int32)

    gi = jax.lax.broadcasted_iota(jnp.int32, (1, _NBLK), 1).astype(jnp.float32)
    be = jnp.zeros((1, _NBLK), jnp.float32)
    for e in range(_E - 1):
        be = be + (gi >= cum_list[e]).astype(jnp.float32)
    blk_ref[...] = be.astype(jnp.int32)


def _expert_kernel(be_ref, xs_ref, w_ref, b_ref, out_ref):
    del be_ref
    acc = jnp.dot(xs_ref[...], w_ref[...], preferred_element_type=jnp.float32)
    out_ref[...] = acc + b_ref[0]


_NC = 2
_NS = 16
_NW = _NC * _NS
_PER = _T // _NW
_CH = 64
_NCH = _PER // _CH


def _vector_mesh():
    return plsc.VectorSubcoreMesh(core_axis_name="c", subcore_axis_name="s")


_SC_SCRATCH = [pltpu.VMEM((_NCH, _CH), jnp.int32),
               pltpu.VMEM((_CH, _D), jnp.float32),
               pltpu.SemaphoreType.DMA]


def _sc_scatter(x, pos):
    @pl.kernel(out_type=jax.ShapeDtypeStruct((_CAP, _D), x.dtype),
               mesh=_vector_mesh(), scratch_types=_SC_SCRATCH)
    def run(x_hbm, i_hbm, o_hbm, idx_v, rows_v, sem):
        wid = jax.lax.axis_index("s") * _NC + jax.lax.axis_index("c")
        base = wid * _PER
        pltpu.sync_copy(i_hbm.at[pl.ds(wid * _NCH, _NCH)], idx_v)
        for t in range(_NCH):
            pltpu.sync_copy(x_hbm.at[pl.ds(base + t * _CH, _CH)], rows_v)
            pltpu.async_copy(rows_v, o_hbm.at[idx_v.at[t]], sem).wait()

    return run(x, pos)


def _sc_gather(src, pos):
    @pl.kernel(out_type=jax.ShapeDtypeStruct((_T, _D), src.dtype),
               mesh=_vector_mesh(), scratch_types=_SC_SCRATCH)
    def run(src_hbm, i_hbm, o_hbm, idx_v, rows_v, sem):
        wid = jax.lax.axis_index("s") * _NC + jax.lax.axis_index("c")
        base = wid * _PER
        pltpu.sync_copy(i_hbm.at[pl.ds(wid * _NCH, _NCH)], idx_v)
        for t in range(_NCH):
            pltpu.async_copy(src_hbm.at[idx_v.at[t]], rows_v, sem).wait()
            pltpu.sync_copy(rows_v, o_hbm.at[pl.ds(base + t * _CH, _CH)])

    return run(src, pos)


@jax.jit
def kernel(x, W_exp, b_exp, W_r1, b_r1, W_r2, b_r2):
    scores = pl.pallas_call(
        _router_kernel,
        grid=(_T // _RB,),
        in_specs=[
            pl.BlockSpec((_RB, _D), lambda i: (i, 0)),
            pl.BlockSpec((_D, 2 * _D), lambda i: (0, 0)),
            pl.BlockSpec((1, 2 * _D), lambda i: (0, 0)),
            pl.BlockSpec((2 * _D, _E), lambda i: (0, 0)),
            pl.BlockSpec((1, _E), lambda i: (0, 0)),
        ],
        out_specs=pl.BlockSpec((_E, _RB), lambda i: (0, i)),
        out_shape=jax.ShapeDtypeStruct((_E, _T), jnp.float32),
    )(x, W_r1, b_r1.reshape(1, -1), W_r2, b_r2.reshape(1, -1))

    s3 = scores.reshape(_E, _C, 128)
    pos2d, blk = pl.pallas_call(
        _dispatch_kernel,
        grid=(1,),
        in_specs=[pl.BlockSpec((_E, _C, 128), lambda i: (0, 0, 0))],
        out_specs=[pl.BlockSpec((_C, 128), lambda i: (0, 0)),
                   pl.BlockSpec((1, _NBLK), lambda i: (0, 0))],
        out_shape=[jax.ShapeDtypeStruct((_C, 128), jnp.int32),
                   jax.ShapeDtypeStruct((1, _NBLK), jnp.int32)],
    )(s3)
    pos = pos2d.reshape(_T // _CH, _CH)
    blk_exp = blk.reshape(_NBLK)

    x_sorted = _sc_scatter(x, pos)

    out_sorted = pl.pallas_call(
        _expert_kernel,
        grid_spec=pltpu.PrefetchScalarGridSpec(
            num_scalar_prefetch=1,
            grid=(_NBLK,),
            in_specs=[
                pl.BlockSpec((_B, _D), lambda g, be: (g, 0)),
                pl.BlockSpec((_D, _O), lambda g, be: (0, be[g])),
                pl.BlockSpec((1, 1, _O), lambda g, be: (be[g], 0, 0)),
            ],
            out_specs=pl.BlockSpec((_B, _O), lambda g, be: (g, 0)),
        ),
        out_shape=jax.ShapeDtypeStruct((_CAP, _O), jnp.float32),
    )(blk_exp, x_sorted, W_exp, b_exp.reshape(_E, 1, _O))

    return _sc_gather(out_sorted, pos)
